# baseline (speedup 1.0000x reference)
import functools

import jax
import jax.numpy as jnp
from jax import lax
from jax.experimental import pallas as pl
from jax.experimental.pallas import tpu as pltpu
from jax.experimental.pallas import tpu_sc as plsc

B = 256
F = 128
L = 16


def _fm_body(idx_hbm, eu_hbm, ei_hbm, ub_hbm, ib_hbm, bias_hbm,
             out_hbm, yfm_hbm, idx_v, eu_v, ei_v, ub_v, ib_v, bias_v,
             yv_v, stage_v, out_v, sem_i, sem_e, sem_b, sem_s):
    c = lax.axis_index("c")
    s = lax.axis_index("s")
    base = s * L

    pltpu.async_copy(idx_hbm.at[s], idx_v, sem_i).wait()
    uidx = idx_v.at[pl.ds(0, L)]
    iidx = idx_v.at[pl.ds(L, L)]
    eu_cp = pltpu.async_copy(eu_hbm.at[uidx], eu_v, sem_e)
    ei_cp = pltpu.async_copy(ei_hbm.at[iidx], ei_v, sem_e)
    ub_cp = pltpu.async_copy(ub_hbm.at[uidx], ub_v, sem_b)
    ib_cp = pltpu.async_copy(ib_hbm.at[iidx], ib_v, sem_b)
    b_cp = pltpu.async_copy(bias_hbm, bias_v, sem_b)
    eu_cp.wait()
    ei_cp.wait()

    iota = lax.iota(jnp.int32, L)

    acc = [jnp.zeros((L,), jnp.float32) for _ in range(4)]
    for f in range(F):
        fcol = jnp.full((L,), f, jnp.int32)
        fu = plsc.load_gather(eu_v, [iota, fcol])
        fi = plsc.load_gather(ei_v, [iota, fcol])
        acc[f % 4] = acc[f % 4] + fu * fi
    yv_v[...] = (acc[0] + acc[1]) + (acc[2] + acc[3])

    pltpu.sync_copy(yv_v, yfm_hbm.at[c, s])

    ub_cp.wait()
    ib_cp.wait()
    b_cp.wait()

    for r in range(8):
        lr = c * 8 + r
        lane = jnp.full((L,), lr, jnp.int32)
        csplat = (plsc.load_gather(ub_v, [lane])
                  + plsc.load_gather(ib_v, [lane]) + bias_v[...])
        for cc in range(B // L):
            if cc < F // L:
                emb = eu_v[lr, pl.ds(cc * L, L)]
            else:
                emb = ei_v[lr, pl.ds((cc - F // L) * L, L)]
            out_v[r, pl.ds(cc * L, L)] = emb + csplat

    plsc.subcore_barrier()
    st_cp = pltpu.async_copy(yfm_hbm.at[c], stage_v, sem_s)
    st_cp.wait()

    for r in range(8):
        for cc in range(B // L):
            yfm_cc = stage_v[cc, pl.ds(0, L)]
            out_v[r, pl.ds(cc * L, L)] = out_v[r, pl.ds(cc * L, L)] + yfm_cc
        if r == 3:
            h0_cp = pltpu.async_copy(
                out_v.at[pl.ds(0, 4)],
                out_hbm.at[pl.ds(base + c * 8, 4)], sem_e)
    h1_cp = pltpu.async_copy(
        out_v.at[pl.ds(4, 4)], out_hbm.at[pl.ds(base + c * 8 + 4, 4)], sem_e)
    h0_cp.wait()
    h1_cp.wait()


@functools.partial(jax.jit, static_argnames=())
def _fm_call(idx, embed_user, embed_item, ub1, ib1, b16):
    mesh = plsc.VectorSubcoreMesh(core_axis_name="c", subcore_axis_name="s")
    run = pl.kernel(
        _fm_body,
        out_type=(jax.ShapeDtypeStruct((B, B), jnp.float32),
                  jax.ShapeDtypeStruct((2, L, L), jnp.float32)),
        mesh=mesh,
        compiler_params=pltpu.CompilerParams(needs_layout_passes=False),
        scratch_types=[
            pltpu.VMEM((2 * L,), jnp.int32),
            pltpu.VMEM((L, F), jnp.float32),
            pltpu.VMEM((L, F), jnp.float32),
            pltpu.VMEM((L,), jnp.float32),
            pltpu.VMEM((L,), jnp.float32),
            pltpu.VMEM((L,), jnp.float32),
            pltpu.VMEM((L,), jnp.float32),
            pltpu.VMEM((L, L), jnp.float32),
            pltpu.VMEM((8, B), jnp.float32),
            pltpu.SemaphoreType.DMA,
            pltpu.SemaphoreType.DMA,
            pltpu.SemaphoreType.DMA,
            pltpu.SemaphoreType.DMA,
        ],
    )
    out, _ = run(idx, embed_user, embed_item, ub1, ib1, b16)
    return out


def kernel(user, item, embed_user, embed_item, u_bias, i_bias, bias_):
    user = user.astype(jnp.int32)
    item = item.astype(jnp.int32)
    idx = jnp.concatenate(
        [user.reshape(L, L), item.reshape(L, L)], axis=1)
    ub1 = u_bias.reshape(-1)
    ib1 = i_bias.reshape(-1)
    b16 = jnp.broadcast_to(bias_, (L,))
    out = _fm_call(idx, embed_user, embed_item, ub1, ib1, b16)
    return out.reshape(-1)


# speedup vs baseline: 1.0011x; 1.0011x over previous
import functools

import jax
import jax.numpy as jnp
from jax import lax
from jax.experimental import pallas as pl
from jax.experimental.pallas import tpu as pltpu
from jax.experimental.pallas import tpu_sc as plsc

B = 256
F = 128
L = 16


def _fm_body(idx_hbm, eu_hbm, ei_hbm, ub_hbm, ib_hbm, bias_hbm,
             out_hbm, yfm_hbm, idx_v, eu_v, ei_v, ub_v, ib_v, bias_v,
             yv_v, stage_v, out_v, sem_i, sem_e, sem_b, sem_s):
    c = lax.axis_index("c")
    s = lax.axis_index("s")
    base = s * L

    pltpu.async_copy(idx_hbm.at[s], idx_v, sem_i).wait()
    uidx = idx_v.at[pl.ds(0, L)]
    iidx = idx_v.at[pl.ds(L, L)]
    eu_cp = pltpu.async_copy(eu_hbm.at[uidx], eu_v, sem_e)
    ei_cp = pltpu.async_copy(ei_hbm.at[iidx], ei_v, sem_e)
    ub_cp = pltpu.async_copy(ub_hbm.at[uidx], ub_v, sem_b)
    ib_cp = pltpu.async_copy(ib_hbm.at[iidx], ib_v, sem_b)
    b_cp = pltpu.async_copy(bias_hbm, bias_v, sem_b)
    eu_cp.wait()
    ei_cp.wait()

    iota = lax.iota(jnp.int32, L)

    acc = [jnp.zeros((L,), jnp.float32) for _ in range(4)]
    for f in range(F):
        fcol = jnp.full((L,), f, jnp.int32)
        fu = plsc.load_gather(eu_v, [iota, fcol])
        fi = plsc.load_gather(ei_v, [iota, fcol])
        acc[f % 4] = acc[f % 4] + fu * fi
    yv_v[...] = (acc[0] + acc[1]) + (acc[2] + acc[3])

    pltpu.sync_copy(yv_v, yfm_hbm.at[c, s])

    ub_cp.wait()
    ib_cp.wait()
    b_cp.wait()

    plsc.subcore_barrier()
    pltpu.async_copy(yfm_hbm.at[c], stage_v, sem_s).wait()

    yfm = [stage_v[cc, pl.ds(0, L)] for cc in range(B // L)]

    for r in range(8):
        lr = c * 8 + r
        lane = jnp.full((L,), lr, jnp.int32)
        csplat = (plsc.load_gather(ub_v, [lane])
                  + plsc.load_gather(ib_v, [lane]) + bias_v[...])
        for cc in range(B // L):
            if cc < F // L:
                emb = eu_v[lr, pl.ds(cc * L, L)]
            else:
                emb = ei_v[lr, pl.ds((cc - F // L) * L, L)]
            out_v[r, pl.ds(cc * L, L)] = emb + (yfm[cc] + csplat)
        if r == 3:
            h0_cp = pltpu.async_copy(
                out_v.at[pl.ds(0, 4)],
                out_hbm.at[pl.ds(base + c * 8, 4)], sem_e)
    h1_cp = pltpu.async_copy(
        out_v.at[pl.ds(4, 4)], out_hbm.at[pl.ds(base + c * 8 + 4, 4)], sem_e)
    h0_cp.wait()
    h1_cp.wait()


@functools.partial(jax.jit, static_argnames=())
def _fm_call(idx, embed_user, embed_item, ub1, ib1, b16):
    mesh = plsc.VectorSubcoreMesh(core_axis_name="c", subcore_axis_name="s")
    run = pl.kernel(
        _fm_body,
        out_type=(jax.ShapeDtypeStruct((B, B), jnp.float32),
                  jax.ShapeDtypeStruct((2, L, L), jnp.float32)),
        mesh=mesh,
        compiler_params=pltpu.CompilerParams(needs_layout_passes=False),
        scratch_types=[
            pltpu.VMEM((2 * L,), jnp.int32),
            pltpu.VMEM((L, F), jnp.float32),
            pltpu.VMEM((L, F), jnp.float32),
            pltpu.VMEM((L,), jnp.float32),
            pltpu.VMEM((L,), jnp.float32),
            pltpu.VMEM((L,), jnp.float32),
            pltpu.VMEM((L,), jnp.float32),
            pltpu.VMEM((L, L), jnp.float32),
            pltpu.VMEM((8, B), jnp.float32),
            pltpu.SemaphoreType.DMA,
            pltpu.SemaphoreType.DMA,
            pltpu.SemaphoreType.DMA,
            pltpu.SemaphoreType.DMA,
        ],
    )
    out, _ = run(idx, embed_user, embed_item, ub1, ib1, b16)
    return out


def kernel(user, item, embed_user, embed_item, u_bias, i_bias, bias_):
    user = user.astype(jnp.int32)
    item = item.astype(jnp.int32)
    idx = jnp.concatenate(
        [user.reshape(L, L), item.reshape(L, L)], axis=1)
    ub1 = u_bias.reshape(-1)
    ib1 = i_bias.reshape(-1)
    b16 = jnp.broadcast_to(bias_, (L,))
    out = _fm_call(idx, embed_user, embed_item, ub1, ib1, b16)
    return out.reshape(-1)


# speedup vs baseline: 1.0262x; 1.0251x over previous
import functools

import jax
import jax.numpy as jnp
from jax import lax
from jax.experimental import pallas as pl
from jax.experimental.pallas import tpu as pltpu
from jax.experimental.pallas import tpu_sc as plsc

B = 256
F = 128
L = 16


def _fm_body(user_hbm, item_hbm, eu_hbm, ei_hbm, ub_hbm, ib_hbm, bias_hbm,
             out_hbm, yfm_hbm, uidx_v, iidx_v, eu_v, ei_v, ub_v, ib_v,
             bias_v, yv_v, stage_v, out_v, sem_i, sem_e, sem_b, sem_s):
    c = lax.axis_index("c")
    s = lax.axis_index("s")
    base = s * L

    u_cp = pltpu.async_copy(user_hbm.at[pl.ds(base, L)], uidx_v, sem_i)
    i_cp = pltpu.async_copy(item_hbm.at[pl.ds(base, L)], iidx_v, sem_i)
    u_cp.wait()
    i_cp.wait()
    uidx = uidx_v
    iidx = iidx_v
    eu_cp = pltpu.async_copy(eu_hbm.at[uidx], eu_v, sem_e)
    ei_cp = pltpu.async_copy(ei_hbm.at[iidx], ei_v, sem_e)
    ub_cp = pltpu.async_copy(ub_hbm.at[uidx], ub_v, sem_b)
    ib_cp = pltpu.async_copy(ib_hbm.at[iidx], ib_v, sem_b)
    b_cp = pltpu.async_copy(bias_hbm, bias_v, sem_b)
    eu_cp.wait()
    ei_cp.wait()

    iota = lax.iota(jnp.int32, L)

    acc = [jnp.zeros((L,), jnp.float32) for _ in range(4)]
    for f in range(F):
        fcol = jnp.full((L,), f, jnp.int32)
        fu = plsc.load_gather(eu_v, [iota, fcol])
        fi = plsc.load_gather(ei_v, [iota, fcol])
        acc[f % 4] = acc[f % 4] + fu * fi
    yv_v[...] = (acc[0] + acc[1]) + (acc[2] + acc[3])

    pltpu.sync_copy(yv_v, yfm_hbm.at[c, s])

    ub_cp.wait()
    ib_cp.wait()
    b_cp.wait()

    plsc.subcore_barrier()
    pltpu.async_copy(yfm_hbm.at[c], stage_v, sem_s).wait()

    yfm = [stage_v[cc, pl.ds(0, L)] for cc in range(B // L)]

    for r in range(8):
        lr = c * 8 + r
        lane = jnp.full((L,), lr, jnp.int32)
        csplat = (plsc.load_gather(ub_v, [lane])
                  + plsc.load_gather(ib_v, [lane]) + bias_v[...])
        for cc in range(B // L):
            if cc < F // L:
                emb = eu_v[lr, pl.ds(cc * L, L)]
            else:
                emb = ei_v[lr, pl.ds((cc - F // L) * L, L)]
            out_v[r, pl.ds(cc * L, L)] = emb + (yfm[cc] + csplat)
        if r == 3:
            h0_cp = pltpu.async_copy(
                out_v.at[pl.ds(0, 4)],
                out_hbm.at[pl.ds(base + c * 8, 4)], sem_e)
    h1_cp = pltpu.async_copy(
        out_v.at[pl.ds(4, 4)], out_hbm.at[pl.ds(base + c * 8 + 4, 4)], sem_e)
    h0_cp.wait()
    h1_cp.wait()


@functools.partial(jax.jit, static_argnames=())
def _fm_call(user, item, embed_user, embed_item, ub1, ib1, b16):
    mesh = plsc.VectorSubcoreMesh(core_axis_name="c", subcore_axis_name="s")
    run = pl.kernel(
        _fm_body,
        out_type=(jax.ShapeDtypeStruct((B, B), jnp.float32),
                  jax.ShapeDtypeStruct((2, L, L), jnp.float32)),
        mesh=mesh,
        compiler_params=pltpu.CompilerParams(needs_layout_passes=False),
        scratch_types=[
            pltpu.VMEM((L,), jnp.int32),
            pltpu.VMEM((L,), jnp.int32),
            pltpu.VMEM((L, F), jnp.float32),
            pltpu.VMEM((L, F), jnp.float32),
            pltpu.VMEM((L,), jnp.float32),
            pltpu.VMEM((L,), jnp.float32),
            pltpu.VMEM((L,), jnp.float32),
            pltpu.VMEM((L,), jnp.float32),
            pltpu.VMEM((L, L), jnp.float32),
            pltpu.VMEM((8, B), jnp.float32),
            pltpu.SemaphoreType.DMA,
            pltpu.SemaphoreType.DMA,
            pltpu.SemaphoreType.DMA,
            pltpu.SemaphoreType.DMA,
        ],
    )
    out, _ = run(user, item, embed_user, embed_item, ub1, ib1, b16)
    return out


def kernel(user, item, embed_user, embed_item, u_bias, i_bias, bias_):
    user = user.astype(jnp.int32)
    item = item.astype(jnp.int32)
    ub1 = u_bias.reshape(-1)
    ib1 = i_bias.reshape(-1)
    b16 = jnp.broadcast_to(bias_, (L,))
    out = _fm_call(user, item, embed_user, embed_item, ub1, ib1, b16)
    return out.reshape(-1)


# speedup vs baseline: 1.0395x; 1.0129x over previous
import functools

import jax
import jax.numpy as jnp
from jax import lax
from jax.experimental import pallas as pl
from jax.experimental.pallas import tpu as pltpu
from jax.experimental.pallas import tpu_sc as plsc

B = 256
F = 128
L = 16


def _fm_body(user_hbm, item_hbm, eu_hbm, ei_hbm, ub_hbm, ib_hbm, bias_hbm,
             out_hbm, yfm_hbm, uidx_v, iidx_v, eu_v, ei_v, ub_v, ib_v,
             bias1_v, yv_v, stage_v, out_v, sem_i, sem_e, sem_b, sem_s):
    c = lax.axis_index("c")
    s = lax.axis_index("s")
    base = s * L

    u_cp = pltpu.async_copy(user_hbm.at[pl.ds(base, L)], uidx_v, sem_i)
    i_cp = pltpu.async_copy(item_hbm.at[pl.ds(base, L)], iidx_v, sem_i)
    u_cp.wait()
    i_cp.wait()
    uidx = uidx_v
    iidx = iidx_v
    eu_cp = pltpu.async_copy(eu_hbm.at[uidx], eu_v, sem_e)
    ei_cp = pltpu.async_copy(ei_hbm.at[iidx], ei_v, sem_e)
    ub_cp = pltpu.async_copy(ub_hbm.at[uidx], ub_v, sem_b)
    ib_cp = pltpu.async_copy(ib_hbm.at[iidx], ib_v, sem_b)
    b_cp = pltpu.async_copy(bias_hbm, bias1_v, sem_b)
    eu_cp.wait()
    ei_cp.wait()

    iota = lax.iota(jnp.int32, L)

    acc = [jnp.zeros((L,), jnp.float32) for _ in range(4)]
    for f in range(F):
        fcol = jnp.full((L,), f, jnp.int32)
        fu = plsc.load_gather(eu_v, [iota, fcol])
        fi = plsc.load_gather(ei_v, [iota, fcol])
        acc[f % 4] = acc[f % 4] + fu * fi
    yv_v[...] = (acc[0] + acc[1]) + (acc[2] + acc[3])

    pltpu.sync_copy(yv_v, yfm_hbm.at[c, s])

    ub_cp.wait()
    ib_cp.wait()
    b_cp.wait()

    plsc.subcore_barrier()
    pltpu.async_copy(yfm_hbm.at[c], stage_v, sem_s).wait()

    yfm = [stage_v[cc, pl.ds(0, L)] for cc in range(B // L)]
    bias_splat = plsc.load_gather(bias1_v, [jnp.zeros((L,), jnp.int32)])

    for r in range(8):
        lr = c * 8 + r
        lane = jnp.full((L,), lr, jnp.int32)
        csplat = (plsc.load_gather(ub_v, [lane])
                  + plsc.load_gather(ib_v, [lane]) + bias_splat)
        for cc in range(B // L):
            if cc < F // L:
                emb = eu_v[lr, pl.ds(cc * L, L)]
            else:
                emb = ei_v[lr, pl.ds((cc - F // L) * L, L)]
            out_v[r, pl.ds(cc * L, L)] = emb + (yfm[cc] + csplat)
        if r == 3:
            h0_cp = pltpu.async_copy(
                out_v.at[pl.ds(0, 4)],
                out_hbm.at[pl.ds(base + c * 8, 4)], sem_e)
    h1_cp = pltpu.async_copy(
        out_v.at[pl.ds(4, 4)], out_hbm.at[pl.ds(base + c * 8 + 4, 4)], sem_e)
    h0_cp.wait()
    h1_cp.wait()


@functools.partial(jax.jit, static_argnames=())
def _fm_call(user, item, embed_user, embed_item, ub1, ib1, b1):
    mesh = plsc.VectorSubcoreMesh(core_axis_name="c", subcore_axis_name="s")
    run = pl.kernel(
        _fm_body,
        out_type=(jax.ShapeDtypeStruct((B, B), jnp.float32),
                  jax.ShapeDtypeStruct((2, L, L), jnp.float32)),
        mesh=mesh,
        compiler_params=pltpu.CompilerParams(needs_layout_passes=False),
        scratch_types=[
            pltpu.VMEM((L,), jnp.int32),
            pltpu.VMEM((L,), jnp.int32),
            pltpu.VMEM((L, F), jnp.float32),
            pltpu.VMEM((L, F), jnp.float32),
            pltpu.VMEM((L,), jnp.float32),
            pltpu.VMEM((L,), jnp.float32),
            pltpu.VMEM((1,), jnp.float32),
            pltpu.VMEM((L,), jnp.float32),
            pltpu.VMEM((L, L), jnp.float32),
            pltpu.VMEM((8, B), jnp.float32),
            pltpu.SemaphoreType.DMA,
            pltpu.SemaphoreType.DMA,
            pltpu.SemaphoreType.DMA,
            pltpu.SemaphoreType.DMA,
        ],
    )
    out, _ = run(user, item, embed_user, embed_item, ub1, ib1, b1)
    return out


def kernel(user, item, embed_user, embed_item, u_bias, i_bias, bias_):
    user = user.astype(jnp.int32)
    item = item.astype(jnp.int32)
    ub1 = u_bias.reshape(-1)
    ib1 = i_bias.reshape(-1)
    out = _fm_call(user, item, embed_user, embed_item, ub1, ib1, bias_)
    return out.reshape(-1)


# speedup vs baseline: 1.0570x; 1.0169x over previous
import functools

import jax
import jax.numpy as jnp
from jax import lax
from jax.experimental import pallas as pl
from jax.experimental.pallas import tpu as pltpu
from jax.experimental.pallas import tpu_sc as plsc

B = 256
F = 128
L = 16


def _fm_body(user_hbm, item_hbm, eu_hbm, ei_hbm, ub_hbm, ib_hbm, bias_hbm,
             out_hbm, yfm_hbm, uidx_v, iidx_v, eu_v, ei_v, ub_v, ib_v,
             bias1_v, yv_v, stage_v, out_v, sem_i, sem_e, sem_b, sem_s):
    c = lax.axis_index("c")
    s = lax.axis_index("s")
    base = s * L

    u_cp = pltpu.async_copy(user_hbm.at[pl.ds(base, L)], uidx_v, sem_i)
    i_cp = pltpu.async_copy(item_hbm.at[pl.ds(base, L)], iidx_v, sem_i)
    u_cp.wait()
    i_cp.wait()
    uidx = uidx_v
    iidx = iidx_v
    eu_cp = pltpu.async_copy(eu_hbm.at[uidx], eu_v, sem_e)
    ei_cp = pltpu.async_copy(ei_hbm.at[iidx], ei_v, sem_e)
    ub_cp = pltpu.async_copy(ub_hbm.at[uidx], ub_v, sem_b)
    ib_cp = pltpu.async_copy(ib_hbm.at[iidx], ib_v, sem_b)
    b_cp = pltpu.async_copy(bias_hbm, bias1_v, sem_b)
    eu_cp.wait()
    ei_cp.wait()

    iota = lax.iota(jnp.int32, L)

    def yfm_step(f, acc):
        a0, a1, a2, a3 = acc
        f4 = 4 * f
        cols = [jnp.full((L,), f4 + k, jnp.int32) for k in range(4)]
        prods = [plsc.load_gather(eu_v, [iota, cols[k]])
                 * plsc.load_gather(ei_v, [iota, cols[k]]) for k in range(4)]
        return (a0 + prods[0], a1 + prods[1], a2 + prods[2], a3 + prods[3])

    zero = jnp.zeros((L,), jnp.float32)
    acc = lax.fori_loop(0, F // 4, yfm_step, (zero, zero, zero, zero))
    yv_v[...] = (acc[0] + acc[1]) + (acc[2] + acc[3])

    pltpu.sync_copy(yv_v, yfm_hbm.at[c, s])

    ub_cp.wait()
    ib_cp.wait()
    b_cp.wait()

    plsc.subcore_barrier()
    pltpu.async_copy(yfm_hbm.at[c], stage_v, sem_s).wait()

    yfm = [stage_v[cc, pl.ds(0, L)] for cc in range(B // L)]
    bias_splat = plsc.load_gather(bias1_v, [jnp.zeros((L,), jnp.int32)])

    def out_row(r, carry):
        lr = c * 8 + r
        lane = jnp.full((L,), lr, jnp.int32)
        csplat = (plsc.load_gather(ub_v, [lane])
                  + plsc.load_gather(ib_v, [lane]) + bias_splat)
        for cc in range(B // L):
            if cc < F // L:
                emb = eu_v[lr, pl.ds(cc * L, L)]
            else:
                emb = ei_v[lr, pl.ds((cc - F // L) * L, L)]
            out_v[r, pl.ds(cc * L, L)] = emb + (yfm[cc] + csplat)
        return carry

    lax.fori_loop(0, 4, out_row, 0)
    h0_cp = pltpu.async_copy(
        out_v.at[pl.ds(0, 4)], out_hbm.at[pl.ds(base + c * 8, 4)], sem_e)
    lax.fori_loop(4, 8, out_row, 0)
    h1_cp = pltpu.async_copy(
        out_v.at[pl.ds(4, 4)], out_hbm.at[pl.ds(base + c * 8 + 4, 4)], sem_e)
    h0_cp.wait()
    h1_cp.wait()


@functools.partial(jax.jit, static_argnames=())
def _fm_call(user, item, embed_user, embed_item, ub1, ib1, b1):
    mesh = plsc.VectorSubcoreMesh(core_axis_name="c", subcore_axis_name="s")
    run = pl.kernel(
        _fm_body,
        out_type=(jax.ShapeDtypeStruct((B, B), jnp.float32),
                  jax.ShapeDtypeStruct((2, L, L), jnp.float32)),
        mesh=mesh,
        compiler_params=pltpu.CompilerParams(needs_layout_passes=False),
        scratch_types=[
            pltpu.VMEM((L,), jnp.int32),
            pltpu.VMEM((L,), jnp.int32),
            pltpu.VMEM((L, F), jnp.float32),
            pltpu.VMEM((L, F), jnp.float32),
            pltpu.VMEM((L,), jnp.float32),
            pltpu.VMEM((L,), jnp.float32),
            pltpu.VMEM((1,), jnp.float32),
            pltpu.VMEM((L,), jnp.float32),
            pltpu.VMEM((L, L), jnp.float32),
            pltpu.VMEM((8, B), jnp.float32),
            pltpu.SemaphoreType.DMA,
            pltpu.SemaphoreType.DMA,
            pltpu.SemaphoreType.DMA,
            pltpu.SemaphoreType.DMA,
        ],
    )
    out, _ = run(user, item, embed_user, embed_item, ub1, ib1, b1)
    return out


def kernel(user, item, embed_user, embed_item, u_bias, i_bias, bias_):
    user = user.astype(jnp.int32)
    item = item.astype(jnp.int32)
    ub1 = u_bias.reshape(-1)
    ib1 = i_bias.reshape(-1)
    out = _fm_call(user, item, embed_user, embed_item, ub1, ib1, bias_)
    return out.reshape(-1)
